# baseline (speedup 1.0000x reference)
import functools

import jax
import jax.numpy as jnp
from jax import lax
from jax.experimental import pallas as pl
from jax.experimental.pallas import tpu as pltpu
from jax.experimental.pallas import tpu_sc as plsc

NC = 2
NS = 16
NW = NC * NS

S = 16384
W = 26
D = 64

SENT_PER_W = S // NW
SPC = 8
CHUNKS = SENT_PER_W // SPC
NBUF = 4


def kernel(input, embedding_weight):
    mesh = plsc.VectorSubcoreMesh(core_axis_name="c", subcore_axis_name="s")

    @functools.partial(
        pl.kernel,
        out_type=jax.ShapeDtypeStruct((S, W, D), jnp.bfloat16),
        mesh=mesh,
        scratch_types=[
            pltpu.VMEM((SENT_PER_W, W), jnp.int32),
            pltpu.VMEM((NBUF, SPC, W, D), jnp.float32),
            pltpu.VMEM((NBUF, SPC, W, D), jnp.bfloat16),
            [pltpu.SemaphoreType.DMA] * NBUF,
            [pltpu.SemaphoreType.DMA] * NBUF,
        ],
        compiler_params=pltpu.CompilerParams(
            needs_layout_passes=False,
            use_tc_tiling_on_sc=False,
            skip_device_barrier=True,
        ),
    )
    def emb(idx_hbm, table_hbm, out_hbm, idx_v, rows_v, out_v, sgs, sos):
        wid = lax.axis_index("s") * NC + lax.axis_index("c")
        sent0 = wid * SENT_PER_W

        pltpu.sync_copy(idx_hbm.at[pl.ds(sent0, SENT_PER_W)], idx_v)

        def fire_gathers(t, b):
            for s in range(SPC):
                pltpu.async_copy(
                    table_hbm.at[idx_v.at[t * SPC + s]],
                    rows_v.at[b, s],
                    sgs[b],
                )

        def wait_gathers(b):
            for s in range(SPC):
                pltpu.make_async_copy(
                    table_hbm.at[pl.ds(0, W)], rows_v.at[b, s], sgs[b]
                ).wait()

        def fire_out(t, b):
            pltpu.async_copy(
                out_v.at[b], out_hbm.at[pl.ds(sent0 + t * SPC, SPC)], sos[b]
            )

        def wait_out(b):
            pltpu.make_async_copy(
                out_hbm.at[pl.ds(0, SPC)], out_v.at[b], sos[b]
            ).wait()

        lane2 = lax.iota(jnp.int32, 16) * 2
        offs = [lane2, lane2 + 1, lane2 + 32, lane2 + 33]

        def convert(b):
            @pl.loop(0, SPC)
            def conv_s(s):
                @pl.loop(0, W, unroll=2)
                def conv_j(j):
                    row = jnp.full((16,), j, dtype=jnp.int32)
                    for h in range(2):
                        evens = plsc.load_gather(
                            rows_v.at[b, s], [row, offs[2 * h]]
                        )
                        odds = plsc.load_gather(
                            rows_v.at[b, s], [row, offs[2 * h + 1]]
                        )
                        out_v[b, s, j, pl.ds(32 * h, 32)] = plsc.pack(
                            evens, odds, format=plsc.PackFormat.INTERLEAVED
                        )

        for b in range(NBUF - 1):
            fire_gathers(b, b)

        @pl.loop(0, CHUNKS, step=NBUF)
        def ring(t0):
            for b in range(NBUF):
                t = t0 + b

                @pl.when(t + (NBUF - 1) < CHUNKS)
                def _():
                    fire_gathers(t + (NBUF - 1), (b + NBUF - 1) % NBUF)

                wait_gathers(b)

                @pl.when(t >= NBUF)
                def _():
                    wait_out(b)

                convert(b)
                fire_out(t, b)

        for b in range(NBUF):
            wait_out(b)

    return emb(input, embedding_weight)


# speedup vs baseline: 1.0113x; 1.0113x over previous
import functools

import jax
import jax.numpy as jnp
from jax import lax
from jax.experimental import pallas as pl
from jax.experimental.pallas import tpu as pltpu
from jax.experimental.pallas import tpu_sc as plsc

NC = 2
NS = 16
NW = NC * NS

S = 16384
W = 26
D = 64

SENT_PER_W = S // NW
SPC = 16
CHUNKS = SENT_PER_W // SPC
RPC = SPC * W


def kernel(input, embedding_weight):
    mesh = plsc.VectorSubcoreMesh(core_axis_name="c", subcore_axis_name="s")

    @functools.partial(
        pl.kernel,
        out_type=jax.ShapeDtypeStruct((S, W, D), jnp.bfloat16),
        mesh=mesh,
        scratch_types=[
            pltpu.VMEM((CHUNKS, RPC), jnp.int32),
            pltpu.VMEM((2, RPC, D), jnp.float32),
            pltpu.VMEM((2, SPC, W, D), jnp.bfloat16),
            pltpu.SemaphoreType.DMA,
            pltpu.SemaphoreType.DMA,
            pltpu.SemaphoreType.DMA,
            pltpu.SemaphoreType.DMA,
        ],
        compiler_params=pltpu.CompilerParams(
            needs_layout_passes=False,
            use_tc_tiling_on_sc=False,
            skip_device_barrier=True,
        ),
    )
    def emb(idx_hbm, table_hbm, out_hbm, idx_v, rows_v, out_v,
            sg0, sg1, so0, so1):
        wid = lax.axis_index("s") * NC + lax.axis_index("c")
        sent0 = wid * SENT_PER_W
        sgs = (sg0, sg1)
        sos = (so0, so1)

        pltpu.sync_copy(idx_hbm.at[pl.ds(wid * CHUNKS, CHUNKS)], idx_v)

        def fire_gather(t, b):
            pltpu.async_copy(table_hbm.at[idx_v.at[t]], rows_v.at[b], sgs[b])

        def wait_gather(b):
            pltpu.make_async_copy(
                table_hbm.at[pl.ds(0, RPC)], rows_v.at[b], sgs[b]
            ).wait()

        def fire_out(t, b):
            pltpu.async_copy(
                out_v.at[b], out_hbm.at[pl.ds(sent0 + t * SPC, SPC)], sos[b]
            )

        def wait_out(b):
            pltpu.make_async_copy(
                out_hbm.at[pl.ds(0, SPC)], out_v.at[b], sos[b]
            ).wait()

        lane2 = lax.iota(jnp.int32, 16) * 2
        offs = [lane2, lane2 + 1, lane2 + 32, lane2 + 33]

        def convert(b):
            @pl.loop(0, SPC)
            def conv_s(s):
                @pl.loop(0, W, unroll=2)
                def conv_j(j):
                    row = jnp.full((16,), s * W + j, dtype=jnp.int32)
                    for h in range(2):
                        evens = plsc.load_gather(
                            rows_v.at[b], [row, offs[2 * h]]
                        )
                        odds = plsc.load_gather(
                            rows_v.at[b], [row, offs[2 * h + 1]]
                        )
                        out_v[b, s, j, pl.ds(32 * h, 32)] = plsc.pack(
                            evens, odds, format=plsc.PackFormat.INTERLEAVED
                        )

        fire_gather(0, 0)

        @pl.loop(0, CHUNKS, step=2)
        def pair(t0):
            for b in range(2):
                t = t0 + b

                @pl.when(t + 1 < CHUNKS)
                def _():
                    fire_gather(t + 1, b ^ 1)

                wait_gather(b)

                @pl.when(t >= 2)
                def _():
                    wait_out(b)

                convert(b)
                fire_out(t, b)

        wait_out(0)
        wait_out(1)

    return emb(input.reshape(S * W // RPC, RPC), embedding_weight)
